# baseline (speedup 1.0000x reference)
import jax
import jax.numpy as jnp
from jax.experimental import pallas as pl

_TOP_K = 128
_MIN_CONF = 0.1
_KEEP = 256
_CBLK = 16
_CHUNK = 64
_SUPER = 16


def _chunkmax_kernel(x_ref, m0_ref, m1_ref, m2_ref, m3_ref):
    x = x_ref[...]
    m0_ref[...] = jnp.max(x[..., 0 * _CHUNK : 1 * _CHUNK], axis=-1)
    m1_ref[...] = jnp.max(x[..., 1 * _CHUNK : 2 * _CHUNK], axis=-1)
    m2_ref[...] = jnp.max(x[..., 2 * _CHUNK : 3 * _CHUNK], axis=-1)
    m3_ref[...] = jnp.max(x[..., 3 * _CHUNK : 4 * _CHUNK], axis=-1)


def _chunk_maxima(points_heatmap, nheat):
    B, C, H, W = points_heatmap.shape
    grid = (B, nheat // _CBLK)
    ospec = pl.BlockSpec((1, _CBLK, H), lambda b, i: (b, i, 0))
    oshape = jax.ShapeDtypeStruct((B, nheat, H), points_heatmap.dtype)
    return pl.pallas_call(
        _chunkmax_kernel,
        grid=grid,
        in_specs=[pl.BlockSpec((1, _CBLK, H, W), lambda b, i: (b, i, 0, 0))],
        out_specs=(ospec, ospec, ospec, ospec),
        out_shape=(oshape, oshape, oshape, oshape),
    )(points_heatmap)


def _narrow(values, unit_ids, keep):
    _, pos = jax.lax.top_k(values, keep)
    bidx = jnp.arange(values.shape[0])[:, None]
    return jnp.sort(unit_ids[bidx, pos], axis=1)


def kernel(points_heatmap):
    B, C, H, W = points_heatmap.shape
    nheat = C - 4
    n_rows = nheat * H
    bidx = jnp.arange(B)[:, None]

    m0, m1, m2, m3 = _chunk_maxima(points_heatmap, nheat)
    chunkmax = jnp.stack([m0, m1, m2, m3], axis=-1).reshape(B, n_rows * 4)
    rowmax = jnp.maximum(jnp.maximum(m0, m1), jnp.maximum(m2, m3)).reshape(
        B, n_rows
    )
    supermax = rowmax.reshape(B, n_rows // _SUPER, _SUPER).max(axis=-1)

    sid = _narrow(
        supermax,
        jnp.broadcast_to(jnp.arange(n_rows // _SUPER), supermax.shape),
        _KEEP,
    )
    cand_rows = (sid[:, :, None] * _SUPER + jnp.arange(_SUPER)).reshape(B, -1)
    rid = _narrow(rowmax[bidx, cand_rows], cand_rows, _KEEP)
    cand_chunks = (rid[:, :, None] * 4 + jnp.arange(4)).reshape(B, -1)
    cid = _narrow(chunkmax[bidx, cand_chunks], cand_chunks, _KEEP)

    heat_chunks = points_heatmap[:, :nheat].reshape(B, n_rows * 4, _CHUNK)
    gathered = heat_chunks[bidx, cid].reshape(B, _KEEP * _CHUNK)
    scores, gpos = jax.lax.top_k(gathered, _TOP_K)

    flat = cid[bidx, gpos // _CHUNK] * _CHUNK + (gpos % _CHUNK)
    clses = (flat // (H * W)).astype(jnp.int32)
    rem = flat % (H * W)
    ys = (rem // W).astype(jnp.int32)
    xs = (rem % W).astype(jnp.int32)

    off_y = points_heatmap[bidx, C - 4, ys, xs]
    off_x = points_heatmap[bidx, C - 3, ys, xs]
    sz_h = points_heatmap[bidx, C - 2, ys, xs]
    sz_w = points_heatmap[bidx, C - 1, ys, xs]

    mask = scores > _MIN_CONF
    obj = jnp.stack(
        [
            ys.astype(jnp.float32) + off_y,
            xs.astype(jnp.float32) + off_x,
            sz_h,
            sz_w,
            clses.astype(jnp.float32),
            scores * mask.astype(jnp.float32),
        ],
        axis=-1,
    )
    return jnp.where(mask[..., None], obj, jnp.zeros_like(obj))


# speedup vs baseline: 1.2717x; 1.2717x over previous
import jax
import jax.numpy as jnp
from jax.experimental import pallas as pl

_TOP_K = 128
_MIN_CONF = 0.1
_KEEP_ROWS = 256
_CBLK = 16


def _rowmax_kernel(x_ref, o_ref):
    o_ref[...] = jnp.max(x_ref[...], axis=-1)


def _row_maxima(points_heatmap, nheat):
    B, C, H, W = points_heatmap.shape
    grid = (B, nheat // _CBLK)
    return pl.pallas_call(
        _rowmax_kernel,
        grid=grid,
        in_specs=[pl.BlockSpec((1, _CBLK, H, W), lambda b, i: (b, i, 0, 0))],
        out_specs=pl.BlockSpec((1, _CBLK, H), lambda b, i: (b, i, 0)),
        out_shape=jax.ShapeDtypeStruct((B, nheat, H), points_heatmap.dtype),
    )(points_heatmap)


def kernel(points_heatmap):
    B, C, H, W = points_heatmap.shape
    nheat = C - 4

    rowmax = _row_maxima(points_heatmap, nheat).reshape(B, nheat * H)

    _, rid = jax.lax.top_k(rowmax, _KEEP_ROWS)
    rid = jnp.sort(rid, axis=1)
    heat_rows = points_heatmap[:, :nheat].reshape(B, nheat * H, W)
    bidx = jnp.arange(B)[:, None]
    gathered = heat_rows[bidx, rid].reshape(B, _KEEP_ROWS * W)
    scores, gpos = jax.lax.top_k(gathered, _TOP_K)

    flat = rid[bidx, gpos // W] * W + (gpos % W)
    clses = (flat // (H * W)).astype(jnp.int32)
    rem = flat % (H * W)
    ys = (rem // W).astype(jnp.int32)
    xs = (rem % W).astype(jnp.int32)

    reg = points_heatmap[:, C - 4 :, :, :].reshape(B, 4, H * W)
    rvals = jnp.take_along_axis(reg, rem[:, None, :], axis=2)
    off_y, off_x, sz_h, sz_w = rvals[:, 0], rvals[:, 1], rvals[:, 2], rvals[:, 3]

    mask = scores > _MIN_CONF
    obj = jnp.stack(
        [
            ys.astype(jnp.float32) + off_y,
            xs.astype(jnp.float32) + off_x,
            sz_h,
            sz_w,
            clses.astype(jnp.float32),
            scores * mask.astype(jnp.float32),
        ],
        axis=-1,
    )
    return jnp.where(mask[..., None], obj, jnp.zeros_like(obj))
